# Initial kernel scaffold; baseline (speedup 1.0000x reference)
import jax
import jax.numpy as jnp
from jax.experimental import pallas as pl


def kernel(weight, alpha, flip_idx):
    raise NotImplementedError("write your pallas kernel here")


# speedup vs baseline: 1.5098x; 1.5098x over previous
import functools

import jax
import jax.numpy as jnp
from jax import lax
from jax.experimental import pallas as pl
from jax.experimental.pallas import tpu as pltpu
from jax.experimental.pallas import tpu_sc as plsc

QN = -127.0
QP = 127.0
MSB = 128

ROWS, COLS = 4096, 4096
BLOCK_ROWS = 256

NUM_WORKERS = 32
LANES = 16
CHUNK = 64
PADDED = NUM_WORKERS * CHUNK


def _dense_body(alpha_ref, w_ref, o_ref):
    a = alpha_ref[0]
    q = jnp.clip(w_ref[...] / a, QN, QP)
    o_ref[...] = jnp.round(q) * a


_dense_quantize = pl.pallas_call(
    _dense_body,
    grid=(ROWS // BLOCK_ROWS,),
    in_specs=[
        pl.BlockSpec(memory_space=pltpu.SMEM),
        pl.BlockSpec((BLOCK_ROWS, COLS), lambda i: (i, 0)),
    ],
    out_specs=pl.BlockSpec((BLOCK_ROWS, COLS), lambda i: (i, 0)),
    out_shape=jax.ShapeDtypeStruct((ROWS, COLS), jnp.float32),
)


def _flip_body(w_hbm, idx_hbm, alpha_hbm, wq_ref, idx_v, w_v, val_v, alpha_v,
               sem):
    cid = lax.axis_index("c")
    sid = lax.axis_index("s")
    wid = sid * 2 + cid
    base = wid * CHUNK
    pltpu.sync_copy(idx_hbm.at[pl.ds(base, CHUNK)], idx_v)
    pltpu.sync_copy(alpha_hbm, alpha_v)
    pltpu.async_copy(w_hbm.at[idx_v], w_v, sem).wait()
    a = alpha_v[...]
    for j in range(CHUNK // LANES):
        w = w_v[pl.ds(j * LANES, LANES)]
        sel = jnp.clip(w / a, QN, QP)
        flipped = (sel.astype(jnp.int32) ^ MSB).astype(jnp.float32)
        val_v[pl.ds(j * LANES, LANES)] = flipped * a
    pltpu.async_copy(val_v, wq_ref.at[idx_v], sem).wait()


_flip_scatter = functools.partial(
    pl.kernel,
    out_type=(),
    mesh=plsc.VectorSubcoreMesh(core_axis_name="c", subcore_axis_name="s"),
    scratch_types=[
        pltpu.VMEM((CHUNK,), jnp.int32),
        pltpu.VMEM((CHUNK,), jnp.float32),
        pltpu.VMEM((CHUNK,), jnp.float32),
        pltpu.VMEM((LANES,), jnp.float32),
        pltpu.SemaphoreType.DMA,
    ],
)(_flip_body)


def kernel(weight, alpha, flip_idx):
    alpha_eff = jnp.maximum(alpha[0], 1e-4)
    wq = _dense_quantize(alpha_eff.reshape(1), weight)
    pad = PADDED - flip_idx.shape[0]
    idx_pad = jnp.concatenate(
        [flip_idx, jnp.broadcast_to(flip_idx[:1], (pad,))])
    alpha_vec = jnp.full((LANES,), alpha_eff, jnp.float32)
    wq_ref = jax.new_ref(wq.reshape(-1))
    _flip_scatter(weight.reshape(-1), idx_pad, alpha_vec, wq_ref)
    return wq_ref[...].reshape(ROWS, COLS)


# speedup vs baseline: 1.5189x; 1.0060x over previous
import jax
import jax.numpy as jnp
from jax import lax
from jax.experimental import pallas as pl
from jax.experimental.pallas import tpu as pltpu
from jax.experimental.pallas import tpu_sc as plsc
from jax._src.pallas import mpmd as _plmpmd

QN = -127.0
QP = 127.0
MSB = 128

ROWS, COLS = 4096, 4096
BLOCK_ROWS = 512

NUM_WORKERS = 32
LANES = 16
CHUNK = 64
PADDED = NUM_WORKERS * CHUNK


def _dense_body(alpha_ref, w_ref, o_ref):
    a = alpha_ref[0]
    q = jnp.clip(w_ref[...] / a, QN, QP)
    o_ref[...] = jnp.round(q) * a


_dense_quantize = pl.pallas_call(
    _dense_body,
    grid=(ROWS // BLOCK_ROWS,),
    in_specs=[
        pl.BlockSpec(memory_space=pltpu.SMEM),
        pl.BlockSpec((BLOCK_ROWS, COLS), lambda i: (i, 0)),
    ],
    out_specs=pl.BlockSpec((BLOCK_ROWS, COLS), lambda i: (i, 0)),
    out_shape=jax.ShapeDtypeStruct((ROWS, COLS), jnp.float32),
)


def _flip_body(w_hbm, idx_hbm, alpha_hbm, wq_in, out_hbm, idx_v, w_v, val_v,
               alpha_v, sem):
    del wq_in
    cid = lax.axis_index("c")
    sid = lax.axis_index("s")
    wid = sid * 2 + cid
    base = wid * CHUNK
    pltpu.sync_copy(idx_hbm.at[pl.ds(base, CHUNK)], idx_v)
    pltpu.sync_copy(alpha_hbm, alpha_v)
    pltpu.async_copy(w_hbm.at[idx_v], w_v, sem).wait()
    a = alpha_v[...]
    for j in range(CHUNK // LANES):
        w = w_v[pl.ds(j * LANES, LANES)]
        sel = jnp.clip(w / a, QN, QP)
        flipped = (sel.astype(jnp.int32) ^ MSB).astype(jnp.float32)
        val_v[pl.ds(j * LANES, LANES)] = flipped * a
    pltpu.async_copy(val_v, out_hbm.at[idx_v], sem).wait()


_flip_scatter = _plmpmd._mpmd_map(
    [(plsc.VectorSubcoreMesh(core_axis_name="c", subcore_axis_name="s"),
      _flip_body)],
    out_types=jax.ShapeDtypeStruct((ROWS * COLS,), jnp.float32),
    input_output_aliases={3: 0},
    scratch_types=[
        pltpu.VMEM((CHUNK,), jnp.int32),
        pltpu.VMEM((CHUNK,), jnp.float32),
        pltpu.VMEM((CHUNK,), jnp.float32),
        pltpu.VMEM((LANES,), jnp.float32),
        pltpu.SemaphoreType.DMA,
    ],
)


def kernel(weight, alpha, flip_idx):
    alpha_eff = jnp.maximum(alpha[0], 1e-4)
    wq = _dense_quantize(alpha_eff.reshape(1), weight)
    pad = PADDED - flip_idx.shape[0]
    idx_pad = jnp.concatenate(
        [flip_idx, jnp.broadcast_to(flip_idx[:1], (pad,))])
    alpha_vec = jnp.full((LANES,), alpha_eff, jnp.float32)
    out = _flip_scatter(weight.reshape(-1), idx_pad, alpha_vec,
                        wq.reshape(-1))
    return out.reshape(ROWS, COLS)
